# baseline (speedup 1.0000x reference)
import jax
import jax.numpy as jnp
from jax import lax
from jax.experimental import pallas as pl
from jax.experimental.pallas import tpu as pltpu
from jax.experimental.pallas import tpu_sc as plsc

D, H, W, C = 2, 200, 176, 128
NNZ = 60000
R = D * H * W
NC, NS = 2, 16

PASSES = 3
CHUNK = 11776
NTRASH = 64
BB = 256
NBATCH = 235
SHORT = NNZ - BB * (NBATCH - 1)
NBI = 15
NPC = BB // 128
ZROWS = (CHUNK + NTRASH) // NS
WB = CHUNK // NS
LASTWB = R - (NC * PASSES - 1) * CHUNK - (NS - 1) * WB


def _sc_body(values, d_hbm, h_hbm, w_hbm, z_hbm, out,
             vals_v, di_v, hi_v, wi_v, idx0, idx1, spm):
    idx_refs = [idx0, idx1]
    cid = lax.axis_index("c")
    tid = lax.axis_index("s")
    iota = lax.iota(jnp.int32, 16)

    def stage_batch(off, nrows):
        pltpu.sync_copy(values.at[pl.ds(off, nrows)], vals_v.at[pl.ds(0, nrows)])
        pltpu.sync_copy(d_hbm.at[pl.ds(off, nrows)], di_v.at[pl.ds(0, nrows)])
        pltpu.sync_copy(h_hbm.at[pl.ds(off, nrows)], hi_v.at[pl.ds(0, nrows)])
        pltpu.sync_copy(w_hbm.at[pl.ds(off, nrows)], wi_v.at[pl.ds(0, nrows)])

    for p in range(PASSES):
        k = cid * PASSES + p
        lo = k * CHUNK
        pltpu.sync_copy(z_hbm, spm.at[pl.ds(tid * ZROWS, ZROWS)])
        plsc.subcore_barrier()

        for i in range(NBI):
            b = tid + NS * i
            off = b * BB
            if i < NBI - 1:
                stage_batch(off, BB)
                limit = jnp.int32(BB)
            else:
                @pl.when(b < NBATCH - 1)
                def _():
                    stage_batch(off, BB)

                @pl.when(b == NBATCH - 1)
                def _():
                    stage_batch(off, SHORT)
                limit = jnp.where(
                    b == NBATCH - 1, jnp.int32(SHORT),
                    jnp.where(b < NBATCH - 1, jnp.int32(BB), jnp.int32(0)))

            for j in range(NPC):
                def grp(gg, _):
                    o = j * 128 + gg * 16
                    dv = di_v[pl.ds(o, 16)]
                    hv = hi_v[pl.ds(o, 16)]
                    wv = wi_v[pl.ds(o, 16)]
                    lin = dv * (H * W) + hv * W + wv
                    ok = (lin >= lo) & (lin < lo + CHUNK) & ((o + iota) < limit)
                    local = jnp.where(ok, lin - lo,
                                      CHUNK + (lin & (NTRASH - 1)))
                    idx_refs[j][pl.ds(gg * 16, 16)] = local
                    return 0

                lax.fori_loop(0, 128 // 16, grp, 0)
                pltpu.sync_copy(vals_v.at[pl.ds(j * 128, 128)],
                                spm.at[idx_refs[j]], add=True)
        plsc.subcore_barrier()

        partial = (k == NC * PASSES - 1) & (tid == NS - 1)

        @pl.when(partial)
        def _():
            pltpu.sync_copy(spm.at[pl.ds(tid * WB, LASTWB)],
                            out.at[pl.ds(lo + tid * WB, LASTWB)])

        @pl.when(~partial)
        def _():
            pltpu.sync_copy(spm.at[pl.ds(tid * WB, WB)],
                            out.at[pl.ds(lo + tid * WB, WB)])
        plsc.subcore_barrier()


def _sc_scatter(values, d_i, h_i, w_i):
    zeros = jnp.zeros((ZROWS, C), jnp.float32)
    mesh = plsc.VectorSubcoreMesh(core_axis_name="c", subcore_axis_name="s")
    return pl.kernel(
        _sc_body,
        out_type=jax.ShapeDtypeStruct((R, C), jnp.float32),
        mesh=mesh,
        scratch_types=[
            pltpu.VMEM((BB, C), jnp.float32),
            pltpu.VMEM((BB,), jnp.int32),
            pltpu.VMEM((BB,), jnp.int32),
            pltpu.VMEM((BB,), jnp.int32),
        ] + [pltpu.VMEM((128,), jnp.int32) for _ in range(NPC)] + [
            pltpu.VMEM_SHARED((CHUNK + NTRASH, C), jnp.float32),
        ],
    )(values, d_i, h_i, w_i, zeros)


TCOLS = 1408


def _tp_body(x_ref, o_ref):
    o_ref[...] = jnp.transpose(x_ref[0], (1, 0))


def _tc_transpose(dense):
    return pl.pallas_call(
        _tp_body,
        grid=(D, (H * W) // TCOLS),
        in_specs=[pl.BlockSpec((1, TCOLS, C), lambda d, i: (d, i, 0))],
        out_specs=pl.BlockSpec((C, TCOLS), lambda d, i: (0, d * ((H * W) // TCOLS) + i)),
        out_shape=jax.ShapeDtypeStruct((C, D * H * W), jnp.float32),
    )(dense)


@jax.jit
def kernel(values, indices_d, indices_h, indices_w):
    values = values.astype(jnp.float32)
    d_i = indices_d.astype(jnp.int32)
    h_i = indices_h.astype(jnp.int32)
    w_i = indices_w.astype(jnp.int32)
    dense = _sc_scatter(values, d_i, h_i, w_i)
    out = _tc_transpose(dense.reshape(D, H * W, C))
    return out.reshape(1, C * D, H, W)


# speedup vs baseline: 1.1846x; 1.1846x over previous
import jax
import jax.numpy as jnp
from jax import lax
from jax.experimental import pallas as pl
from jax.experimental.pallas import tpu as pltpu
from jax.experimental.pallas import tpu_sc as plsc

D, H, W, C = 2, 200, 176, 128
NNZ = 60000
R = D * H * W
NC, NS = 2, 16

PASSES = 3
CHUNK = 11776
NTRASH = 64
BB = 256
NBATCH = 235
SHORT = NNZ - BB * (NBATCH - 1)
NBI = 15
NPC = BB // 128
ZROWS = (CHUNK + NTRASH) // NS
WB = CHUNK // NS
LASTWB = R - (NC * PASSES - 1) * CHUNK - (NS - 1) * WB


def _sc_body(values, d_hbm, h_hbm, w_hbm, z_hbm, out,
             vals_v, di_v, hi_v, wi_v, idx0, idx1, spm):
    idx_refs = [idx0, idx1]
    cid = lax.axis_index("c")
    tid = lax.axis_index("s")
    iota = lax.iota(jnp.int32, 16)

    def stage_batch(off, nrows):
        pltpu.sync_copy(values.at[pl.ds(off, nrows)], vals_v.at[pl.ds(0, nrows)])
        pltpu.sync_copy(d_hbm.at[pl.ds(off, nrows)], di_v.at[pl.ds(0, nrows)])
        pltpu.sync_copy(h_hbm.at[pl.ds(off, nrows)], hi_v.at[pl.ds(0, nrows)])
        pltpu.sync_copy(w_hbm.at[pl.ds(off, nrows)], wi_v.at[pl.ds(0, nrows)])

    for p in range(PASSES):
        k = cid * PASSES + p
        lo = k * CHUNK
        pltpu.sync_copy(z_hbm, spm.at[pl.ds(tid * ZROWS, ZROWS)])
        plsc.subcore_barrier()

        for i in range(NBI):
            b = tid + NS * i
            off = b * BB
            if i < NBI - 1:
                stage_batch(off, BB)
                limit = jnp.int32(BB)
            else:
                @pl.when(b < NBATCH - 1)
                def _():
                    stage_batch(off, BB)

                @pl.when(b == NBATCH - 1)
                def _():
                    stage_batch(off, SHORT)
                limit = jnp.where(
                    b == NBATCH - 1, jnp.int32(SHORT),
                    jnp.where(b < NBATCH - 1, jnp.int32(BB), jnp.int32(0)))

            for j in range(NPC):
                def grp(gg, _):
                    o = j * 128 + gg * 16
                    dv = di_v[pl.ds(o, 16)]
                    hv = hi_v[pl.ds(o, 16)]
                    wv = wi_v[pl.ds(o, 16)]
                    lin = dv * (H * W) + hv * W + wv
                    ok = (lin >= lo) & (lin < lo + CHUNK) & ((o + iota) < limit)
                    local = jnp.where(ok, lin - lo,
                                      CHUNK + (lin & (NTRASH - 1)))
                    idx_refs[j][pl.ds(gg * 16, 16)] = local
                    return 0

                lax.fori_loop(0, 128 // 16, grp, 0)
                pltpu.sync_copy(vals_v.at[pl.ds(j * 128, 128)],
                                spm.at[idx_refs[j]], add=True)
        plsc.subcore_barrier()

        partial = (k == NC * PASSES - 1) & (tid == NS - 1)

        @pl.when(partial)
        def _():
            pltpu.sync_copy(spm.at[pl.ds(tid * WB, LASTWB)],
                            out.at[pl.ds(lo + tid * WB, LASTWB)])

        @pl.when(~partial)
        def _():
            pltpu.sync_copy(spm.at[pl.ds(tid * WB, WB)],
                            out.at[pl.ds(lo + tid * WB, WB)])
        plsc.subcore_barrier()


def _sc_scatter(values, d_i, h_i, w_i):
    zeros = jnp.zeros((ZROWS, C), jnp.float32)
    mesh = plsc.VectorSubcoreMesh(core_axis_name="c", subcore_axis_name="s")
    return pl.kernel(
        _sc_body,
        out_type=jax.ShapeDtypeStruct((R, C), jnp.float32),
        mesh=mesh,
        scratch_types=[
            pltpu.VMEM((BB, C), jnp.float32),
            pltpu.VMEM((BB,), jnp.int32),
            pltpu.VMEM((BB,), jnp.int32),
            pltpu.VMEM((BB,), jnp.int32),
        ] + [pltpu.VMEM((128,), jnp.int32) for _ in range(NPC)] + [
            pltpu.VMEM_SHARED((CHUNK + NTRASH, C), jnp.float32),
        ],
    )(values, d_i, h_i, w_i, zeros)


RB = 440

import numpy as _np
_PE = _np.zeros((C, C * D), _np.float32)
_PO = _np.zeros((C, C * D), _np.float32)
_PE[_np.arange(C), 2 * _np.arange(C)] = 1.0
_PO[_np.arange(C), 2 * _np.arange(C) + 1] = 1.0


def _il_body(x_ref, pe_ref, po_ref, o_ref):
    o_ref[...] = (
        jnp.dot(x_ref[0], pe_ref[...], preferred_element_type=jnp.float32)
        + jnp.dot(x_ref[1], po_ref[...], preferred_element_type=jnp.float32))


def _tc_interleave(dense):
    return pl.pallas_call(
        _il_body,
        grid=((H * W) // RB,),
        in_specs=[
            pl.BlockSpec((D, RB, C), lambda i: (0, i, 0)),
            pl.BlockSpec((C, C * D), lambda i: (0, 0)),
            pl.BlockSpec((C, C * D), lambda i: (0, 0)),
        ],
        out_specs=pl.BlockSpec((RB, C * D), lambda i: (i, 0)),
        out_shape=jax.ShapeDtypeStruct((H * W, C * D), jnp.float32),
    )(dense, jnp.asarray(_PE), jnp.asarray(_PO))


@jax.jit
def kernel(values, indices_d, indices_h, indices_w):
    values = values.astype(jnp.float32)
    d_i = indices_d.astype(jnp.int32)
    h_i = indices_h.astype(jnp.int32)
    w_i = indices_w.astype(jnp.int32)
    dense = _sc_scatter(values, d_i, h_i, w_i)
    out = _tc_interleave(dense.reshape(D, H * W, C))
    return jnp.transpose(out.reshape(H, W, C * D), (2, 0, 1))[None]


# speedup vs baseline: 1.6550x; 1.3971x over previous
import jax
import jax.numpy as jnp
from jax import lax
from jax.experimental import pallas as pl
from jax.experimental.pallas import tpu as pltpu
from jax.experimental.pallas import tpu_sc as plsc

D, H, W, C = 2, 200, 176, 128
NNZ = 60000
R = D * H * W
NC, NS = 2, 16

PASSES = 3
CHUNK = 11776
NTRASH = 64
BB = 128
NBATCH = 469
SHORT = NNZ - BB * (NBATCH - 1)
NBI = 30
ZROWS = (CHUNK + NTRASH) // NS
WB = CHUNK // NS
LASTWB = R - (NC * PASSES - 1) * CHUNK - (NS - 1) * WB


def _sc_body(values, d_hbm, h_hbm, w_hbm, z_hbm, out,
             vals_v0, vals_v1, di_v0, di_v1, hi_v0, hi_v1, wi_v0, wi_v1,
             idx0, idx1, sem0, sem1, spm):
    vals_b = [vals_v0, vals_v1]
    di_b = [di_v0, di_v1]
    hi_b = [hi_v0, hi_v1]
    wi_b = [wi_v0, wi_v1]
    idx_b = [idx0, idx1]
    sem_b = [sem0, sem1]
    cid = lax.axis_index("c")
    tid = lax.axis_index("s")
    iota = lax.iota(jnp.int32, 16)

    def start_stage(i, u):
        off = (tid + NS * i) * BB
        return [
            pltpu.async_copy(values.at[pl.ds(off, BB)], vals_b[u], sem_b[u]),
            pltpu.async_copy(d_hbm.at[pl.ds(off, BB)], di_b[u], sem_b[u]),
            pltpu.async_copy(h_hbm.at[pl.ds(off, BB)], hi_b[u], sem_b[u]),
            pltpu.async_copy(w_hbm.at[pl.ds(off, BB)], wi_b[u], sem_b[u]),
        ]

    def stage_sync(off, nrows, u):
        pltpu.sync_copy(values.at[pl.ds(off, nrows)],
                        vals_b[u].at[pl.ds(0, nrows)])
        pltpu.sync_copy(d_hbm.at[pl.ds(off, nrows)], di_b[u].at[pl.ds(0, nrows)])
        pltpu.sync_copy(h_hbm.at[pl.ds(off, nrows)], hi_b[u].at[pl.ds(0, nrows)])
        pltpu.sync_copy(w_hbm.at[pl.ds(off, nrows)], wi_b[u].at[pl.ds(0, nrows)])

    def process(u, lo, limit):
        def grp(gg, _):
            o = gg * 16
            dv = di_b[u][pl.ds(o, 16)]
            hv = hi_b[u][pl.ds(o, 16)]
            wv = wi_b[u][pl.ds(o, 16)]
            lin = dv * (H * W) + hv * W + wv
            ok = (lin >= lo) & (lin < lo + CHUNK) & ((o + iota) < limit)
            local = jnp.where(ok, lin - lo, CHUNK + (lin & (NTRASH - 1)))
            idx_b[u][pl.ds(o, 16)] = local
            return 0

        lax.fori_loop(0, BB // 16, grp, 0)
        pltpu.sync_copy(vals_b[u], spm.at[idx_b[u]], add=True)

    for p in range(PASSES):
        k = cid * PASSES + p
        lo = k * CHUNK
        pltpu.sync_copy(z_hbm, spm.at[pl.ds(tid * ZROWS, ZROWS)])
        plsc.subcore_barrier()

        descs = start_stage(0, 0)
        for i in range(NBI - 1):
            u = i % 2
            nxt = None
            if i + 1 < NBI - 1:
                nxt = start_stage(i + 1, 1 - u)
            for dsc in descs:
                dsc.wait()
            process(u, lo, jnp.int32(BB))
            descs = nxt
        b = tid + NS * (NBI - 1)
        u = (NBI - 1) % 2

        @pl.when(b < NBATCH - 1)
        def _():
            stage_sync(b * BB, BB, u)

        @pl.when(b == NBATCH - 1)
        def _():
            stage_sync(b * BB, SHORT, u)
        limit = jnp.where(
            b == NBATCH - 1, jnp.int32(SHORT),
            jnp.where(b < NBATCH - 1, jnp.int32(BB), jnp.int32(0)))
        process(u, lo, limit)
        plsc.subcore_barrier()

        partial = (k == NC * PASSES - 1) & (tid == NS - 1)

        @pl.when(partial)
        def _():
            pltpu.sync_copy(spm.at[pl.ds(tid * WB, LASTWB)],
                            out.at[pl.ds(lo + tid * WB, LASTWB)])

        @pl.when(~partial)
        def _():
            pltpu.sync_copy(spm.at[pl.ds(tid * WB, WB)],
                            out.at[pl.ds(lo + tid * WB, WB)])
        plsc.subcore_barrier()


def _sc_scatter(values, d_i, h_i, w_i):
    zeros = jnp.zeros((ZROWS, C), jnp.float32)
    mesh = plsc.VectorSubcoreMesh(core_axis_name="c", subcore_axis_name="s")
    return pl.kernel(
        _sc_body,
        out_type=jax.ShapeDtypeStruct((R, C), jnp.float32),
        mesh=mesh,
        scratch_types=[
            pltpu.VMEM((BB, C), jnp.float32),
            pltpu.VMEM((BB, C), jnp.float32),
            pltpu.VMEM((BB,), jnp.int32),
            pltpu.VMEM((BB,), jnp.int32),
            pltpu.VMEM((BB,), jnp.int32),
            pltpu.VMEM((BB,), jnp.int32),
            pltpu.VMEM((BB,), jnp.int32),
            pltpu.VMEM((BB,), jnp.int32),
            pltpu.VMEM((128,), jnp.int32),
            pltpu.VMEM((128,), jnp.int32),
            pltpu.SemaphoreType.DMA,
            pltpu.SemaphoreType.DMA,
            pltpu.VMEM_SHARED((CHUNK + NTRASH, C), jnp.float32),
        ],
    )(values, d_i, h_i, w_i, zeros)


RB = 440

import numpy as _np
_PE = _np.zeros((C, C * D), _np.float32)
_PO = _np.zeros((C, C * D), _np.float32)
_PE[_np.arange(C), 2 * _np.arange(C)] = 1.0
_PO[_np.arange(C), 2 * _np.arange(C) + 1] = 1.0


def _il_body(x_ref, pe_ref, po_ref, o_ref):
    o_ref[...] = (
        jnp.dot(x_ref[0], pe_ref[...], preferred_element_type=jnp.float32)
        + jnp.dot(x_ref[1], po_ref[...], preferred_element_type=jnp.float32))


def _tc_interleave(dense):
    return pl.pallas_call(
        _il_body,
        grid=((H * W) // RB,),
        in_specs=[
            pl.BlockSpec((D, RB, C), lambda i: (0, i, 0)),
            pl.BlockSpec((C, C * D), lambda i: (0, 0)),
            pl.BlockSpec((C, C * D), lambda i: (0, 0)),
        ],
        out_specs=pl.BlockSpec((RB, C * D), lambda i: (i, 0)),
        out_shape=jax.ShapeDtypeStruct((H * W, C * D), jnp.float32),
    )(dense, jnp.asarray(_PE), jnp.asarray(_PO))


@jax.jit
def kernel(values, indices_d, indices_h, indices_w):
    values = values.astype(jnp.float32)
    d_i = indices_d.astype(jnp.int32)
    h_i = indices_h.astype(jnp.int32)
    w_i = indices_w.astype(jnp.int32)
    dense = _sc_scatter(values, d_i, h_i, w_i)
    out = _tc_interleave(dense.reshape(D, H * W, C))
    return jnp.transpose(out.reshape(H, W, C * D), (2, 0, 1))[None]
